# Initial kernel scaffold; baseline (speedup 1.0000x reference)
import jax
import jax.numpy as jnp
from jax.experimental import pallas as pl


def kernel(x, pos, pe_w, pe_b, g_norm, wq, wk, wv, wc1k, wc2k, wc1v, wc2v, wg, wo):
    raise NotImplementedError("write your pallas kernel here")


# speedup vs baseline: 4.7933x; 4.7933x over previous
import functools

import jax
import jax.numpy as jnp
import numpy as np
from jax.experimental import pallas as pl
from jax.experimental.pallas import tpu as pltpu

DIM = 1024; HEADS = 16; KV_HEADS = 4; GQ = HEADS // KV_HEADS; DH = 128
BALL = 256; SW = 32; CB = 32; CS = 16; FB = 32; NFB = BALL // FB
NUM_W = (BALL - CB) // CS + 1
NUM_WP = 16
N_TOK = 8192; POSD = 3
SCALE = DH ** -0.5
NB = N_TOK // BALL

_ov = np.zeros((NUM_WP, NFB), np.float32)
for _w in range(NUM_W):
    _ws, _we = _w * CS, _w * CS + CB
    for _fb in range(NFB):
        _fs, _fe = _fb * FB, _fb * FB + FB
        _ov[_w, _fb] = max(0, min(_we, _fe) - max(_ws, _fs)) / float(CB)
_OVL = jnp.asarray(_ov)

_inv = 1.0 / (10000.0 ** (np.arange(0, DH, 2, dtype=np.float32) / DH))
_ang = np.arange(BALL, dtype=np.float32)[:, None] * _inv[None, :]
_COS = jnp.asarray(np.cos(_ang))
_SIN = jnp.asarray(np.sin(_ang))

_PERM = np.concatenate([np.arange(0, DH, 2), np.arange(1, DH, 2)])

_NEG = -1e9


def _rot_half(x, cos, sin):
    x1 = x[:, : DH // 2]
    x2 = x[:, DH // 2:]
    return jnp.concatenate([x1 * cos - x2 * sin, x1 * sin + x2 * cos], axis=1)


def _softmax(s):
    m = jnp.max(s, axis=-1, keepdims=True)
    e = jnp.exp(s - m)
    return e / jnp.sum(e, axis=-1, keepdims=True)


def _nsa_ball_kernel(x_ref, pos_ref, pe_w_ref, pe_b_ref, g_norm_ref,
                     wq_ref, wk_ref, wv_ref,
                     wc1k_ref, wc2k_ref, wc1v_ref, wc2v_ref,
                     wg_ref, wo_ref, ovl_ref, cos_ref, sin_ref, out_ref):
    f32 = jnp.float32
    dot = functools.partial(jnp.dot, preferred_element_type=f32)

    p = pos_ref[:]
    rel = p - jnp.mean(p, axis=0, keepdims=True)
    xb = x_ref[:] + dot(rel, pe_w_ref[:]) + pe_b_ref[:]
    xn = xb * jax.lax.rsqrt(jnp.mean(xb * xb, axis=-1, keepdims=True) + 1e-6)
    xn = xn * g_norm_ref[:]

    q = dot(xn, wq_ref[:])
    k = dot(xn, wk_ref[:])
    v = dot(xn, wv_ref[:])
    gates = jax.nn.sigmoid(dot(xn, wg_ref[:]))

    cos = cos_ref[:]
    sin = sin_ref[:]
    ovl = ovl_ref[:]

    t_col = jax.lax.broadcasted_iota(jnp.int32, (BALL, 1), 0)
    w_row = jax.lax.broadcasted_iota(jnp.int32, (1, NUM_WP), 1)
    cmask = (w_row * CS + CB - 1) <= t_col
    fb_row = jax.lax.broadcasted_iota(jnp.int32, (1, NFB), 1)
    curblk = t_col // FB
    fmask = fb_row < curblk
    col = jax.lax.broadcasted_iota(jnp.int32, (1, BALL), 1)
    colb = col // FB
    causal = col <= t_col
    dtm = t_col - col
    smask = (dtm >= 0) & (dtm < SW)

    outs = []
    for h in range(KV_HEADS):
        k_h = k[:, h * DH:(h + 1) * DH]
        v_h = v[:, h * DH:(h + 1) * DH]
        kr_h = _rot_half(k_h, cos, sin)

        k16 = k_h.reshape(NUM_WP, CS * DH)
        v16 = v_h.reshape(NUM_WP, CS * DH)
        k16n = pltpu.roll(k16, NUM_WP - 1, 0)
        v16n = pltpu.roll(v16, NUM_WP - 1, 0)
        kwin = jnp.concatenate([k16, k16n], axis=1)
        vwin = jnp.concatenate([v16, v16n], axis=1)
        ck = dot(jnp.maximum(dot(kwin, wc1k_ref[h]), 0.0), wc2k_ref[h])
        cv = dot(jnp.maximum(dot(vwin, wc1v_ref[h]), 0.0), wc2v_ref[h])

        acs, ocs, qrs = [], [], []
        for g in range(GQ):
            hh = h * GQ + g
            qr = _rot_half(q[:, hh * DH:(hh + 1) * DH], cos, sin)
            qrs.append(qr)
            sc = dot(qr, ck.T) * SCALE
            ac = _softmax(jnp.where(cmask, sc, _NEG))
            acs.append(ac)
            oc = dot(ac, cv)
            ocs.append(jnp.where(t_col >= CB - 1, oc, 0.0))

        imp = (acs[0] + acs[1] + acs[2] + acs[3]) * 0.25
        fimp = jnp.where(fmask, dot(imp, ovl), _NEG)
        best = fimp[:, 0:1]
        sel = jnp.zeros((BALL, 1), jnp.int32)
        for j in range(1, NFB):
            cand = fimp[:, j:j + 1]
            better = cand > best
            sel = jnp.where(better, j, sel)
            best = jnp.where(better, cand, best)
        mask_f = causal & ((colb == curblk) | (colb == sel))

        for g in range(GQ):
            hh = h * GQ + g
            s = dot(qrs[g], kr_h.T) * SCALE
            a_f = _softmax(jnp.where(mask_f, s, _NEG))
            o_f = dot(a_f, v_h)
            a_s = _softmax(jnp.where(smask, s, _NEG))
            o_s = dot(a_s, v_h)
            g0 = gates[:, 0 * HEADS + hh][:, None]
            g1 = gates[:, 1 * HEADS + hh][:, None]
            g2 = gates[:, 2 * HEADS + hh][:, None]
            outs.append(g0 * ocs[g] + g1 * o_f + g2 * o_s)

    cat = jnp.concatenate(outs, axis=1)
    out_ref[:] = dot(cat, wo_ref[:])


def kernel(x, pos, pe_w, pe_b, g_norm, wq, wk, wv,
           wc1k, wc2k, wc1v, wc2v, wg, wo):
    perm = jnp.asarray(_PERM)
    wq_p = wq.reshape(DIM, HEADS, DH)[:, :, perm].reshape(DIM, HEADS * DH)
    wk_p = wk.reshape(DIM, KV_HEADS, DH)[:, :, perm].reshape(DIM, KV_HEADS * DH)
    wc1k_p = wc1k.reshape(KV_HEADS, CB, DH, DH)[:, :, perm, :].reshape(
        KV_HEADS, CB * DH, DH)
    wc2k_p = wc2k[:, :, perm]

    full = lambda *shape: pl.BlockSpec(shape, lambda b: (0,) * len(shape))
    grid_spec = pl.GridSpec(
        grid=(NB,),
        in_specs=[
            pl.BlockSpec((BALL, DIM), lambda b: (b, 0)),
            pl.BlockSpec((BALL, POSD), lambda b: (b, 0)),
            full(POSD, DIM),
            full(1, DIM),
            full(1, DIM),
            full(DIM, HEADS * DH),
            full(DIM, KV_HEADS * DH),
            full(DIM, KV_HEADS * DH),
            full(KV_HEADS, CB * DH, DH),
            full(KV_HEADS, DH, DH),
            full(KV_HEADS, CB * DH, DH),
            full(KV_HEADS, DH, DH),
            full(DIM, 3 * HEADS),
            full(HEADS * DH, DIM),
            full(NUM_WP, NFB),
            full(BALL, DH // 2),
            full(BALL, DH // 2),
        ],
        out_specs=pl.BlockSpec((BALL, DIM), lambda b: (b, 0)),
    )
    return pl.pallas_call(
        _nsa_ball_kernel,
        grid_spec=grid_spec,
        out_shape=jax.ShapeDtypeStruct((N_TOK, DIM), jnp.float32),
    )(x, pos, pe_w, pe_b[None, :], g_norm[None, :],
      wq_p, wk_p, wv, wc1k_p, wc2k_p, wc1v, wc2v, wg, wo,
      _OVL, _COS, _SIN)


# speedup vs baseline: 5.0939x; 1.0627x over previous
import functools

import jax
import jax.numpy as jnp
import numpy as np
from jax.experimental import pallas as pl
from jax.experimental.pallas import tpu as pltpu

DIM = 1024; HEADS = 16; KV_HEADS = 4; GQ = HEADS // KV_HEADS; DH = 128
BALL = 256; SW = 32; CB = 32; CS = 16; FB = 32; NFB = BALL // FB
NUM_W = (BALL - CB) // CS + 1
NUM_WP = 16
N_TOK = 8192; POSD = 3
SCALE = DH ** -0.5
NB = N_TOK // BALL

_ov = np.zeros((NUM_WP, NFB), np.float32)
for _w in range(NUM_W):
    _ws, _we = _w * CS, _w * CS + CB
    for _fb in range(NFB):
        _fs, _fe = _fb * FB, _fb * FB + FB
        _ov[_w, _fb] = max(0, min(_we, _fe) - max(_ws, _fs)) / float(CB)
_OVL = jnp.asarray(_ov)

_inv = 1.0 / (10000.0 ** (np.arange(0, DH, 2, dtype=np.float32) / DH))
_ang = np.arange(BALL, dtype=np.float32)[:, None] * _inv[None, :]
_COS = jnp.asarray(np.cos(_ang))
_SIN = jnp.asarray(np.sin(_ang))

_PERM = np.concatenate([np.arange(0, DH, 2), np.arange(1, DH, 2)])

_NEG = -1e9


def _rot_half(x, cos, sin):
    x1 = x[:, : DH // 2]
    x2 = x[:, DH // 2:]
    return jnp.concatenate([x1 * cos - x2 * sin, x1 * sin + x2 * cos], axis=1)


def _softmax(s):
    m = jnp.max(s, axis=-1, keepdims=True)
    e = jnp.exp(s - m)
    return e / jnp.sum(e, axis=-1, keepdims=True)


def _nsa_ball_kernel(x_ref, pos_ref, pe_w_ref, pe_b_ref, g_norm_ref,
                     wq_ref, wk_ref, wv_ref,
                     wc1k_ref, wc2k_ref, wc1v_ref, wc2v_ref,
                     wg_ref, wo_ref, ovl_ref, cos_ref, sin_ref, out_ref):
    f32 = jnp.float32
    bf16 = jnp.bfloat16
    dot = functools.partial(jnp.dot, preferred_element_type=f32)
    dott = lambda a, b: jax.lax.dot_general(
        a, b, (((1,), (1,)), ((), ())), preferred_element_type=f32)

    p = pos_ref[:]
    rel = p - jnp.mean(p, axis=0, keepdims=True)
    xb = x_ref[:] + dot(rel, pe_w_ref[:]) + pe_b_ref[:]
    xn = xb * jax.lax.rsqrt(jnp.mean(xb * xb, axis=-1, keepdims=True) + 1e-6)
    xn = xn * g_norm_ref[:]

    xnb = xn.astype(bf16)
    q = dot(xn, wq_ref[:])
    k = dot(xn, wk_ref[:])
    v = dot(xnb, wv_ref[:])
    gates = jax.nn.sigmoid(dot(xn, wg_ref[:]))

    cos = cos_ref[:]
    sin = sin_ref[:]
    ovl = ovl_ref[:]

    t_col = jax.lax.broadcasted_iota(jnp.int32, (BALL, 1), 0)
    w_row = jax.lax.broadcasted_iota(jnp.int32, (1, NUM_WP), 1)
    cmask = (w_row * CS + CB - 1) <= t_col
    fb_row = jax.lax.broadcasted_iota(jnp.int32, (1, NFB), 1)
    curblk = t_col // FB
    fmask = fb_row < curblk
    col = jax.lax.broadcasted_iota(jnp.int32, (1, BALL), 1)
    colb = col // FB
    causal = col <= t_col
    dtm = t_col - col
    smask = (dtm >= 0) & (dtm < SW)

    outs = []
    for h in range(KV_HEADS):
        k_h = k[:, h * DH:(h + 1) * DH]
        v_h = v[:, h * DH:(h + 1) * DH]
        kr_h = _rot_half(k_h, cos, sin)

        k16 = k_h.reshape(NUM_WP, CS * DH)
        v16 = v_h.reshape(NUM_WP, CS * DH)
        k16n = pltpu.roll(k16, NUM_WP - 1, 0)
        v16n = pltpu.roll(v16, NUM_WP - 1, 0)
        kwin = jnp.concatenate([k16, k16n], axis=1)
        vwin = jnp.concatenate([v16, v16n], axis=1).astype(bf16)
        ck = dot(jnp.maximum(dot(kwin, wc1k_ref[h]), 0.0), wc2k_ref[h])
        cv = dot(jnp.maximum(dot(vwin, wc1v_ref[h]), 0.0).astype(bf16),
                 wc2v_ref[h])
        cvb = cv.astype(bf16)
        v_hb = v_h.astype(bf16)

        acs, ocs, qrs = [], [], []
        for g in range(GQ):
            hh = h * GQ + g
            qr = _rot_half(q[:, hh * DH:(hh + 1) * DH], cos, sin)
            qrs.append(qr)
            sc = dott(qr, ck) * SCALE
            ac = _softmax(jnp.where(cmask, sc, _NEG))
            acs.append(ac)
            oc = dot(ac.astype(bf16), cvb)
            ocs.append(jnp.where(t_col >= CB - 1, oc, 0.0))

        imp = (acs[0] + acs[1] + acs[2] + acs[3]) * 0.25
        fimp = jnp.where(fmask, dot(imp, ovl), _NEG)
        best = fimp[:, 0:1]
        sel = jnp.zeros((BALL, 1), jnp.int32)
        for j in range(1, NFB):
            cand = fimp[:, j:j + 1]
            better = cand > best
            sel = jnp.where(better, j, sel)
            best = jnp.where(better, cand, best)
        mask_f = causal & ((colb == curblk) | (colb == sel))

        for g in range(GQ):
            hh = h * GQ + g
            s = dott(qrs[g], kr_h) * SCALE
            a_f = _softmax(jnp.where(mask_f, s, _NEG))
            o_f = dot(a_f.astype(bf16), v_hb)
            a_s = _softmax(jnp.where(smask, s, _NEG))
            o_s = dot(a_s.astype(bf16), v_hb)
            g0 = gates[:, 0 * HEADS + hh][:, None]
            g1 = gates[:, 1 * HEADS + hh][:, None]
            g2 = gates[:, 2 * HEADS + hh][:, None]
            outs.append(g0 * ocs[g] + g1 * o_f + g2 * o_s)

    cat = jnp.concatenate(outs, axis=1).astype(bf16)
    out_ref[:] = dot(cat, wo_ref[:])


def kernel(x, pos, pe_w, pe_b, g_norm, wq, wk, wv,
           wc1k, wc2k, wc1v, wc2v, wg, wo):
    perm = jnp.asarray(_PERM)
    wq_p = wq.reshape(DIM, HEADS, DH)[:, :, perm].reshape(DIM, HEADS * DH)
    wk_p = wk.reshape(DIM, KV_HEADS, DH)[:, :, perm].reshape(DIM, KV_HEADS * DH)
    wc1k_p = wc1k.reshape(KV_HEADS, CB, DH, DH)[:, :, perm, :].reshape(
        KV_HEADS, CB * DH, DH)
    wc2k_p = wc2k[:, :, perm]

    bf = jnp.bfloat16
    wv_b = wv.astype(bf)
    wc1v_b = wc1v.astype(bf); wc2v_b = wc2v.astype(bf)
    wo_b = wo.astype(bf)

    full = lambda *shape: pl.BlockSpec(shape, lambda b: (0,) * len(shape))
    grid_spec = pl.GridSpec(
        grid=(NB,),
        in_specs=[
            pl.BlockSpec((BALL, DIM), lambda b: (b, 0)),
            pl.BlockSpec((BALL, POSD), lambda b: (b, 0)),
            full(POSD, DIM),
            full(1, DIM),
            full(1, DIM),
            full(DIM, HEADS * DH),
            full(DIM, KV_HEADS * DH),
            full(DIM, KV_HEADS * DH),
            full(KV_HEADS, CB * DH, DH),
            full(KV_HEADS, DH, DH),
            full(KV_HEADS, CB * DH, DH),
            full(KV_HEADS, DH, DH),
            full(DIM, 3 * HEADS),
            full(HEADS * DH, DIM),
            full(NUM_WP, NFB),
            full(BALL, DH // 2),
            full(BALL, DH // 2),
        ],
        out_specs=pl.BlockSpec((BALL, DIM), lambda b: (b, 0)),
    )
    return pl.pallas_call(
        _nsa_ball_kernel,
        grid_spec=grid_spec,
        out_shape=jax.ShapeDtypeStruct((N_TOK, DIM), jnp.float32),
    )(x, pos, pe_w, pe_b[None, :], g_norm[None, :],
      wq_p, wk_p, wv_b, wc1k_p, wc2k_p, wc1v_b, wc2v_b, wg, wo_b,
      _OVL, _COS, _SIN)


# speedup vs baseline: 6.9199x; 1.3585x over previous
import functools

import jax
import jax.numpy as jnp
import numpy as np
from jax.experimental import pallas as pl
from jax.experimental.pallas import tpu as pltpu

DIM = 1024; HEADS = 16; KV_HEADS = 4; GQ = HEADS // KV_HEADS; DH = 128
BALL = 256; SW = 32; CB = 32; CS = 16; FB = 32; NFB = BALL // FB
NUM_W = (BALL - CB) // CS + 1
NUM_WP = 16
N_TOK = 8192; POSD = 3
SCALE = DH ** -0.5
NB = N_TOK // BALL

_ov = np.zeros((NUM_WP, NFB), np.float32)
for _w in range(NUM_W):
    _ws, _we = _w * CS, _w * CS + CB
    for _fb in range(NFB):
        _fs, _fe = _fb * FB, _fb * FB + FB
        _ov[_w, _fb] = max(0, min(_we, _fe) - max(_ws, _fs)) / float(CB)
_OVL = jnp.asarray(_ov)

_inv = 1.0 / (10000.0 ** (np.arange(0, DH, 2, dtype=np.float32) / DH))
_ang = np.arange(BALL, dtype=np.float32)[:, None] * _inv[None, :]
_COS = jnp.asarray(np.cos(_ang))
_SIN = jnp.asarray(np.sin(_ang))

_PERM = np.concatenate([np.arange(0, DH, 2), np.arange(1, DH, 2)])

_NEG = -1e9


def _rot_half(x, cos, sin):
    x1 = x[:, : DH // 2]
    x2 = x[:, DH // 2:]
    return jnp.concatenate([x1 * cos - x2 * sin, x1 * sin + x2 * cos], axis=1)


def _softmax(s):
    m = jnp.max(s, axis=-1, keepdims=True)
    e = jnp.exp(s - m)
    return e / jnp.sum(e, axis=-1, keepdims=True)


def _nsa_ball_kernel(x_ref, pos_ref, pe_w_ref, pe_b_ref, g_norm_ref,
                     wq_ref, wk_ref, wv_ref,
                     wc1k_ref, wc2k_ref, wc1v_ref, wc2v_ref,
                     wg_ref, wo_ref, ovl_ref, cos_ref, sin_ref, out_ref):
    f32 = jnp.float32
    bf16 = jnp.bfloat16
    dot = functools.partial(jnp.dot, preferred_element_type=f32)
    dott = lambda a, b: jax.lax.dot_general(
        a, b, (((1,), (1,)), ((), ())), preferred_element_type=f32)

    p = pos_ref[:]
    rel = p - jnp.mean(p, axis=0, keepdims=True)
    xb = x_ref[:] + dot(rel, pe_w_ref[:]) + pe_b_ref[:]
    xn = xb * jax.lax.rsqrt(jnp.mean(xb * xb, axis=-1, keepdims=True) + 1e-6)
    xn = xn * g_norm_ref[:]

    xnb = xn.astype(bf16)
    q = dot(xn, wq_ref[:])
    k = dot(xn, wk_ref[:])
    v = dot(xnb, wv_ref[:])
    gates = jax.nn.sigmoid(dot(xn, wg_ref[:]))

    cos = cos_ref[:]
    sin = sin_ref[:]
    ovl = ovl_ref[:]

    B4 = GQ * BALL
    t4 = jax.lax.broadcasted_iota(jnp.int32, (B4, 1), 0) & (BALL - 1)
    w_row = jax.lax.broadcasted_iota(jnp.int32, (1, NUM_WP), 1)
    mask_c = w_row * CS + CB - 1 <= t4
    hasc = jnp.where(t4 >= CB - 1, 1.0, 0.0)
    col = jax.lax.broadcasted_iota(jnp.int32, (1, BALL), 1)
    colb = col // FB
    curblk4 = t4 // FB
    causal4 = col <= t4
    ccur4 = colb == curblk4
    dtm = t4 - col
    bias_s = jnp.where((dtm >= 0) & (dtm < SW), 0.0, _NEG)
    t1 = jax.lax.broadcasted_iota(jnp.int32, (BALL, 1), 0)
    fb_row = jax.lax.broadcasted_iota(jnp.int32, (1, NFB), 1)
    fmask = fb_row < t1 // FB
    ones8 = jnp.ones((BALL, 8), bf16)

    outs = [None] * HEADS
    for h in range(KV_HEADS):
        k_h = k[:, h * DH:(h + 1) * DH]
        v_h = v[:, h * DH:(h + 1) * DH]
        kr_h = _rot_half(k_h, cos, sin)
        v_ext = jnp.concatenate([v_h.astype(bf16), ones8], axis=1)

        k16 = k_h.reshape(NUM_WP, CS * DH)
        v16 = v_h.reshape(NUM_WP, CS * DH)
        k16n = pltpu.roll(k16, NUM_WP - 1, 0)
        v16n = pltpu.roll(v16, NUM_WP - 1, 0)
        kwin = jnp.concatenate([k16, k16n], axis=1)
        vwin = jnp.concatenate([v16, v16n], axis=1).astype(bf16)
        ck = dot(jnp.maximum(dot(kwin, wc1k_ref[h]), 0.0), wc2k_ref[h])
        cv = dot(jnp.maximum(dot(vwin, wc1v_ref[h]), 0.0).astype(bf16),
                 wc2v_ref[h])
        cvb = cv.astype(bf16)

        q4 = jnp.concatenate(
            [_rot_half(q[:, (h * GQ + g) * DH:(h * GQ + g + 1) * DH],
                       cos, sin) for g in range(GQ)], axis=0)

        s_c = jnp.where(mask_c, dott(q4, ck) * SCALE, _NEG)
        e_c = jnp.exp(s_c - jnp.max(s_c, axis=-1, keepdims=True))
        ac = e_c / jnp.sum(e_c, axis=-1, keepdims=True)
        oc = dot(ac.astype(bf16), cvb) * hasc

        imp = ((ac[0:BALL] + ac[BALL:2 * BALL])
               + (ac[2 * BALL:3 * BALL] + ac[3 * BALL:]))
        fimp = jnp.where(fmask, dot(imp, ovl), _NEG)
        best = fimp[:, 0:1]
        sel = jnp.zeros((BALL, 1), jnp.int32)
        for j in range(1, NFB):
            cand = fimp[:, j:j + 1]
            better = cand > best
            sel = jnp.where(better, j, sel)
            best = jnp.where(better, cand, best)
        sel4 = jnp.concatenate([sel] * GQ, axis=0)
        bias_f = jnp.where(causal4 & (ccur4 | (colb == sel4)), 0.0, _NEG)

        s4 = dott(q4, kr_h) * SCALE
        e_f = jnp.exp(s4 + bias_f).astype(bf16)
        e_s = jnp.exp(s4 + bias_s).astype(bf16)
        of2 = dot(e_f, v_ext)
        os2 = dot(e_s, v_ext)
        rof = 1.0 / of2[:, DH:DH + 1]
        ros = 1.0 / os2[:, DH:DH + 1]

        for g in range(GQ):
            hh = h * GQ + g
            r = slice(g * BALL, (g + 1) * BALL)
            g0 = gates[:, 0 * HEADS + hh][:, None]
            g1 = gates[:, 1 * HEADS + hh][:, None]
            g2 = gates[:, 2 * HEADS + hh][:, None]
            outs[hh] = (g0 * oc[r] + (g1 * rof[r]) * of2[r, :DH]
                        + (g2 * ros[r]) * os2[r, :DH])

    cat = jnp.concatenate(outs, axis=1).astype(bf16)
    out_ref[:] = dot(cat, wo_ref[:])


def kernel(x, pos, pe_w, pe_b, g_norm, wq, wk, wv,
           wc1k, wc2k, wc1v, wc2v, wg, wo):
    perm = jnp.asarray(_PERM)
    wq_p = wq.reshape(DIM, HEADS, DH)[:, :, perm].reshape(DIM, HEADS * DH)
    wk_p = wk.reshape(DIM, KV_HEADS, DH)[:, :, perm].reshape(DIM, KV_HEADS * DH)
    wc1k_p = wc1k.reshape(KV_HEADS, CB, DH, DH)[:, :, perm, :].reshape(
        KV_HEADS, CB * DH, DH)
    wc2k_p = wc2k[:, :, perm]

    bf = jnp.bfloat16
    wv_b = wv.astype(bf)
    wc1v_b = wc1v.astype(bf); wc2v_b = wc2v.astype(bf)
    wo_b = wo.astype(bf)

    full = lambda *shape: pl.BlockSpec(shape, lambda b: (0,) * len(shape))
    grid_spec = pl.GridSpec(
        grid=(NB,),
        in_specs=[
            pl.BlockSpec((BALL, DIM), lambda b: (b, 0)),
            pl.BlockSpec((BALL, POSD), lambda b: (b, 0)),
            full(POSD, DIM),
            full(1, DIM),
            full(1, DIM),
            full(DIM, HEADS * DH),
            full(DIM, KV_HEADS * DH),
            full(DIM, KV_HEADS * DH),
            full(KV_HEADS, CB * DH, DH),
            full(KV_HEADS, DH, DH),
            full(KV_HEADS, CB * DH, DH),
            full(KV_HEADS, DH, DH),
            full(DIM, 3 * HEADS),
            full(HEADS * DH, DIM),
            full(NUM_WP, NFB),
            full(BALL, DH // 2),
            full(BALL, DH // 2),
        ],
        out_specs=pl.BlockSpec((BALL, DIM), lambda b: (b, 0)),
    )
    return pl.pallas_call(
        _nsa_ball_kernel,
        grid_spec=grid_spec,
        out_shape=jax.ShapeDtypeStruct((N_TOK, DIM), jnp.float32),
    )(x, pos, pe_w, pe_b[None, :], g_norm[None, :],
      wq_p, wk_p, wv_b, wc1k_p, wc2k_p, wc1v_b, wc2v_b, wg, wo_b,
      _OVL, _COS, _SIN)


# speedup vs baseline: 7.0253x; 1.0152x over previous
import functools

import jax
import jax.numpy as jnp
import numpy as np
from jax.experimental import pallas as pl
from jax.experimental.pallas import tpu as pltpu

DIM = 1024; HEADS = 16; KV_HEADS = 4; GQ = HEADS // KV_HEADS; DH = 128
BALL = 256; SW = 32; CB = 32; CS = 16; FB = 32; NFB = BALL // FB
NUM_W = (BALL - CB) // CS + 1
NUM_WP = 16
N_TOK = 8192; POSD = 3
SCALE = DH ** -0.5
NB = N_TOK // BALL

_ov = np.zeros((NUM_WP, NFB), np.float32)
for _w in range(NUM_W):
    _ws, _we = _w * CS, _w * CS + CB
    for _fb in range(NFB):
        _fs, _fe = _fb * FB, _fb * FB + FB
        _ov[_w, _fb] = max(0, min(_we, _fe) - max(_ws, _fs)) / float(CB)
_OVL = jnp.asarray(_ov)

def _rotary_tables():
    inv = 1.0 / (10000.0 ** (jnp.arange(0, DH, 2, dtype=jnp.float32) / DH))
    ang = jnp.arange(BALL, dtype=jnp.float32)[:, None] * inv[None, :]
    return jnp.cos(ang), jnp.sin(ang)

_PERM = np.concatenate([np.arange(0, DH, 2), np.arange(1, DH, 2)])

_NEG = -1e9


def _rot_half(x, cos, sin):
    x1 = x[:, : DH // 2]
    x2 = x[:, DH // 2:]
    return jnp.concatenate([x1 * cos - x2 * sin, x1 * sin + x2 * cos], axis=1)


def _softmax(s):
    m = jnp.max(s, axis=-1, keepdims=True)
    e = jnp.exp(s - m)
    return e / jnp.sum(e, axis=-1, keepdims=True)


def _nsa_ball_kernel(x_ref, pos_ref, pe_w_ref, pe_b_ref, g_norm_ref,
                     wq_ref, wk_ref, wv_ref,
                     wc1k_ref, wc2k_ref, wc1v_ref, wc2v_ref,
                     wg_ref, wo_ref, ovl_ref, cos_ref, sin_ref, out_ref):
    f32 = jnp.float32
    bf16 = jnp.bfloat16
    dot = functools.partial(jnp.dot, preferred_element_type=f32)
    dott = lambda a, b: jax.lax.dot_general(
        a, b, (((1,), (1,)), ((), ())), preferred_element_type=f32)

    p = pos_ref[:]
    rel = p - jnp.mean(p, axis=0, keepdims=True)
    xb = x_ref[:] + dot(rel, pe_w_ref[:]) + pe_b_ref[:]
    xn = xb * jax.lax.rsqrt(jnp.mean(xb * xb, axis=-1, keepdims=True) + 1e-6)
    xn = xn * g_norm_ref[:]

    q = dot(xn, wq_ref[:])
    k = dot(xn, wk_ref[:])
    v = dot(xn, wv_ref[:])
    gates = jax.nn.sigmoid(dot(xn, wg_ref[:]))

    cos = cos_ref[:]
    sin = sin_ref[:]
    ovl = ovl_ref[:]

    B4 = GQ * BALL
    t4 = jax.lax.broadcasted_iota(jnp.int32, (B4, 1), 0) & (BALL - 1)
    w_row = jax.lax.broadcasted_iota(jnp.int32, (1, NUM_WP), 1)
    mask_c = w_row * CS + CB - 1 <= t4
    hasc = jnp.where(t4 >= CB - 1, 1.0, 0.0)
    col = jax.lax.broadcasted_iota(jnp.int32, (1, BALL), 1)
    colb = col // FB
    curblk4 = t4 // FB
    causal4 = col <= t4
    ccur4 = colb == curblk4
    dtm = t4 - col
    bias_s = jnp.where((dtm >= 0) & (dtm < SW), 0.0, _NEG)
    t1 = jax.lax.broadcasted_iota(jnp.int32, (BALL, 1), 0)
    fb_row = jax.lax.broadcasted_iota(jnp.int32, (1, NFB), 1)
    fmask = fb_row < t1 // FB
    ones8 = jnp.ones((BALL, 8), f32)

    outs = [None] * HEADS
    for h in range(KV_HEADS):
        k_h = k[:, h * DH:(h + 1) * DH]
        v_h = v[:, h * DH:(h + 1) * DH]
        kr_h = _rot_half(k_h, cos, sin)
        v_ext = jnp.concatenate([v_h, ones8], axis=1)

        k16 = k_h.reshape(NUM_WP, CS * DH)
        v16 = v_h.reshape(NUM_WP, CS * DH)
        k16n = pltpu.roll(k16, NUM_WP - 1, 0)
        v16n = pltpu.roll(v16, NUM_WP - 1, 0)
        kwin = jnp.concatenate([k16, k16n], axis=1)
        vwin = jnp.concatenate([v16, v16n], axis=1)
        ck = dot(jnp.maximum(dot(kwin, wc1k_ref[h]), 0.0), wc2k_ref[h])
        cv = dot(jnp.maximum(dot(vwin, wc1v_ref[h]), 0.0), wc2v_ref[h])

        q4 = jnp.concatenate(
            [_rot_half(q[:, (h * GQ + g) * DH:(h * GQ + g + 1) * DH],
                       cos, sin) for g in range(GQ)], axis=0)

        s_c = jnp.where(mask_c, dott(q4, ck) * SCALE, _NEG)
        e_c = jnp.exp(s_c - jnp.max(s_c, axis=-1, keepdims=True))
        ac = e_c / jnp.sum(e_c, axis=-1, keepdims=True)
        oc = dot(ac, cv) * hasc

        imp = ((ac[0:BALL] + ac[BALL:2 * BALL])
               + (ac[2 * BALL:3 * BALL] + ac[3 * BALL:]))
        fimp = jnp.where(fmask, dot(imp, ovl), _NEG)
        best = fimp[:, 0:1]
        sel = jnp.zeros((BALL, 1), jnp.int32)
        for j in range(1, NFB):
            cand = fimp[:, j:j + 1]
            better = cand > best
            sel = jnp.where(better, j, sel)
            best = jnp.where(better, cand, best)
        sel4 = jnp.concatenate([sel] * GQ, axis=0)
        bias_f = jnp.where(causal4 & (ccur4 | (colb == sel4)), 0.0, _NEG)

        s4 = dott(q4, kr_h) * SCALE
        e_f = jnp.exp(s4 + bias_f)
        e_s = jnp.exp(s4 + bias_s)
        of2 = dot(e_f, v_ext)
        os2 = dot(e_s, v_ext)
        rof = 1.0 / of2[:, DH:DH + 1]
        ros = 1.0 / os2[:, DH:DH + 1]

        for g in range(GQ):
            hh = h * GQ + g
            r = slice(g * BALL, (g + 1) * BALL)
            g0 = gates[:, 0 * HEADS + hh][:, None]
            g1 = gates[:, 1 * HEADS + hh][:, None]
            g2 = gates[:, 2 * HEADS + hh][:, None]
            outs[hh] = (g0 * oc[r] + (g1 * rof[r]) * of2[r, :DH]
                        + (g2 * ros[r]) * os2[r, :DH])

    cat = jnp.concatenate(outs, axis=1)
    out_ref[:] = dot(cat, wo_ref[:])


def kernel(x, pos, pe_w, pe_b, g_norm, wq, wk, wv,
           wc1k, wc2k, wc1v, wc2v, wg, wo):
    perm = jnp.asarray(_PERM)
    wq_p = wq.reshape(DIM, HEADS, DH)[:, :, perm].reshape(DIM, HEADS * DH)
    wk_p = wk.reshape(DIM, KV_HEADS, DH)[:, :, perm].reshape(DIM, KV_HEADS * DH)
    wc1k_p = wc1k.reshape(KV_HEADS, CB, DH, DH)[:, :, perm, :].reshape(
        KV_HEADS, CB * DH, DH)
    wc2k_p = wc2k[:, :, perm]

    wv_b = wv
    wc1v_b = wc1v; wc2v_b = wc2v
    wo_b = wo

    full = lambda *shape: pl.BlockSpec(shape, lambda b: (0,) * len(shape))
    grid_spec = pl.GridSpec(
        grid=(NB,),
        in_specs=[
            pl.BlockSpec((BALL, DIM), lambda b: (b, 0)),
            pl.BlockSpec((BALL, POSD), lambda b: (b, 0)),
            full(POSD, DIM),
            full(1, DIM),
            full(1, DIM),
            full(DIM, HEADS * DH),
            full(DIM, KV_HEADS * DH),
            full(DIM, KV_HEADS * DH),
            full(KV_HEADS, CB * DH, DH),
            full(KV_HEADS, DH, DH),
            full(KV_HEADS, CB * DH, DH),
            full(KV_HEADS, DH, DH),
            full(DIM, 3 * HEADS),
            full(HEADS * DH, DIM),
            full(NUM_WP, NFB),
            full(BALL, DH // 2),
            full(BALL, DH // 2),
        ],
        out_specs=pl.BlockSpec((BALL, DIM), lambda b: (b, 0)),
    )
    cos_t, sin_t = _rotary_tables()
    return pl.pallas_call(
        _nsa_ball_kernel,
        grid_spec=grid_spec,
        out_shape=jax.ShapeDtypeStruct((N_TOK, DIM), jnp.float32),
    )(x, pos, pe_w, pe_b[None, :], g_norm[None, :],
      wq_p, wk_p, wv_b, wc1k_p, wc2k_p, wc1v_b, wc2v_b, wg, wo_b,
      _OVL, cos_t, sin_t)
